# Initial kernel scaffold; baseline (speedup 1.0000x reference)
import jax
import jax.numpy as jnp
from jax.experimental import pallas as pl


def kernel(x, edge_index, edge_attr, batch, frag_x, frag_edge_index, frag_edge_attr, frag_batch, junction_edge_index, junction_edge_attr, junction_batch, motif_nodes, params):
    raise NotImplementedError("write your pallas kernel here")


# speedup vs baseline: 8.4777x; 8.4777x over previous
import functools
import jax
import jax.numpy as jnp
from jax import lax
from jax.experimental import pallas as pl
from jax.experimental.pallas import tpu as pltpu
from jax.experimental.pallas import tpu_sc as plsc

_INTERPRET = False

F32 = jnp.float32
HW = 128
SC_CHUNK = 128
SC_WORKERS = 32


def _lrelu(x, s=0.01):
    return jnp.where(x >= 0, x, s * x)


def _elu(x):
    return jnp.where(x > 0, x, jnp.exp(jnp.minimum(x, 0.0)) - 1.0)


def _dot(a, b):
    return jnp.dot(a, b, preferred_element_type=F32)


def _dotx(a, b):
    return jnp.dot(a, b, preferred_element_type=F32,
                   precision=lax.Precision.HIGHEST)


def _bn_from_stats(y, st, nrows):
    mu = st[0:1, :] / nrows
    var = st[1:2, :] / nrows - mu * mu
    return (y - mu) * lax.rsqrt(var + 1e-5)


def _stats(X, W, b, tm):
    M, D = X.shape
    Do = W.shape[1]
    b2 = b.reshape(1, Do)

    def body(x_ref, w_ref, b_ref, o_ref):
        i = pl.program_id(0)

        @pl.when(i == 0)
        def _():
            o_ref[...] = jnp.zeros_like(o_ref)

        y = _dot(x_ref[...], w_ref[...]) + b_ref[...]
        o_ref[0:1, :] += jnp.sum(y, axis=0, keepdims=True)
        o_ref[1:2, :] += jnp.sum(y * y, axis=0, keepdims=True)

    return pl.pallas_call(
        body, grid=(M // tm,),
        in_specs=[pl.BlockSpec((tm, D), lambda i: (i, 0)),
                  pl.BlockSpec((D, Do), lambda i: (0, 0)),
                  pl.BlockSpec((1, Do), lambda i: (0, 0))],
        out_specs=pl.BlockSpec((2, Do), lambda i: (0, 0)),
        out_shape=jax.ShapeDtypeStruct((2, Do), F32),
        interpret=_INTERPRET,
    )(X, W, b2)


def _echain(ea, W0, b0, stats, nrows, hparams, tm):
    E, Din = ea.shape
    has_embed = stats is not None
    flat = []
    for (peW, peb, ae0, ae1) in hparams:
        flat += [peW, peb.reshape(1, -1), ae0.reshape(-1, 1), ae1.reshape(-1, 1)]

    def body(*refs):
        ea_ref = refs[0]
        if has_embed:
            st, w0r, b0r = refs[1], refs[2], refs[3]
            hp = refs[4:12]
            outs = refs[12:]
            y = _dot(ea_ref[...], w0r[...]) + b0r[...]
            xb = _lrelu(_bn_from_stats(y, st[...], nrows))
        else:
            hp = refs[1:9]
            outs = refs[9:]
            xb = ea_ref[...]
        for h in range(2):
            peW, peb, ae0, ae1 = hp[4 * h:4 * h + 4]
            eh = _lrelu(_dot(xb, peW[...]) + peb[...])
            outs[2 * h][...] = _dot(eh, ae0[...])
            outs[2 * h + 1][...] = _dot(eh, ae1[...])

    D2 = flat[0].shape[0]
    in_specs = [pl.BlockSpec((tm, Din), lambda i: (i, 0))]
    args = [ea]
    if has_embed:
        in_specs += [pl.BlockSpec((2, W0.shape[1]), lambda i: (0, 0)),
                     pl.BlockSpec(W0.shape, lambda i: (0, 0)),
                     pl.BlockSpec((1, W0.shape[1]), lambda i: (0, 0))]
        args += [stats, W0, b0.reshape(1, -1)]
    for t in flat:
        in_specs.append(pl.BlockSpec(t.shape, lambda i: (0, 0)))
        args.append(t)
    outs = pl.pallas_call(
        body, grid=(E // tm,),
        in_specs=in_specs,
        out_specs=[pl.BlockSpec((tm, 1), lambda i: (i, 0))] * 4,
        out_shape=[jax.ShapeDtypeStruct((E, 1), F32)] * 4,
        interpret=_INTERPRET,
    )(*args)
    return [o.reshape(E) for o in outs]


def _embed_heads(X, W0, b0, stats, nrows, heads, tm):
    N, D = X.shape
    has_bn = stats is not None
    flat = []
    for (pnW, pnb, asv, adv, mW, mb) in heads:
        flat += [pnW, pnb.reshape(1, -1), asv.reshape(-1, 1),
                 adv.reshape(-1, 1), mW, mb.reshape(1, -1)]

    def body(*refs):
        x_ref = refs[0]
        if has_bn:
            st, w0r, b0r = refs[1], refs[2], refs[3]
            hp = refs[4:16]
            outs = refs[16:]
            ex = _lrelu(_bn_from_stats(_dot(x_ref[...], w0r[...]) + b0r[...],
                                       st[...], nrows))
        else:
            hp = refs[1:13]
            outs = refs[13:]
            ex = x_ref[...]
        for h in range(2):
            pnW, pnb, asv, adv, mW, mb = hp[6 * h:6 * h + 6]
            h0 = _lrelu(_dot(ex, pnW[...]) + pnb[...])
            outs[4 * h][...] = h0
            outs[4 * h + 1][...] = _dot(h0, mW[...]) + mb[...]
            outs[4 * h + 2][...] = _dot(h0, asv[...])
            outs[4 * h + 3][...] = _dot(h0, adv[...])

    in_specs = [pl.BlockSpec((tm, D), lambda i: (i, 0))]
    args = [X]
    if has_bn:
        in_specs += [pl.BlockSpec((2, W0.shape[1]), lambda i: (0, 0)),
                     pl.BlockSpec(W0.shape, lambda i: (0, 0)),
                     pl.BlockSpec((1, W0.shape[1]), lambda i: (0, 0))]
        args += [stats, W0, b0.reshape(1, -1)]
    for t in flat:
        in_specs.append(pl.BlockSpec(t.shape, lambda i: (0, 0)))
        args.append(t)
    out_specs, out_shape = [], []
    for _ in range(2):
        out_specs += [pl.BlockSpec((tm, HW), lambda i: (i, 0)),
                      pl.BlockSpec((tm, HW), lambda i: (i, 0)),
                      pl.BlockSpec((tm, 1), lambda i: (i, 0)),
                      pl.BlockSpec((tm, 1), lambda i: (i, 0))]
        out_shape += [jax.ShapeDtypeStruct((N, HW), F32),
                      jax.ShapeDtypeStruct((N, HW), F32),
                      jax.ShapeDtypeStruct((N, 1), F32),
                      jax.ShapeDtypeStruct((N, 1), F32)]
    o = pl.pallas_call(body, grid=(N // tm,), in_specs=in_specs,
                       out_specs=out_specs, out_shape=out_shape,
                       interpret=_INTERPRET)(*args)
    return [(o[0], o[1], o[2].reshape(N), o[3].reshape(N)),
            (o[4], o[5], o[6].reshape(N), o[7].reshape(N))]


def _node_update(h_prev, acc, zp, nxt, tm):
    N = h_prev.shape[0]

    def body(*refs):
        h_ref, a_ref, z_ref = refs[:3]
        a = a_ref[0] + a_ref[1]
        z = jnp.sum(z_ref[...], axis=1, keepdims=True)
        msg = a / (z + 1e-16)
        hn = _elu(h_ref[...] + msg)
        if nxt is None:
            refs[3][...] = hn
        else:
            asv, adv, mW, mb = refs[3], refs[4], refs[5], refs[6]
            refs[7][...] = hn
            refs[8][...] = _dot(hn, mW[...]) + mb[...]
            refs[9][...] = _dot(hn, asv[...])
            refs[10][...] = _dot(hn, adv[...])

    in_specs = [pl.BlockSpec((tm, HW), lambda i: (i, 0)),
                pl.BlockSpec((2, tm, HW), lambda i: (0, i, 0)),
                pl.BlockSpec((tm, SC_WORKERS), lambda i: (i, 0))]
    args = [h_prev, acc, zp]
    if nxt is None:
        out_specs = pl.BlockSpec((tm, 128), lambda i: (i, 0))
        out_shape = jax.ShapeDtypeStruct((N, 128), F32)
    else:
        asv, adv, mW, mb = nxt
        in_specs += [pl.BlockSpec((128, 1), lambda i: (0, 0)),
                     pl.BlockSpec((128, 1), lambda i: (0, 0)),
                     pl.BlockSpec(mW.shape, lambda i: (0, 0)),
                     pl.BlockSpec((1, 128), lambda i: (0, 0))]
        args += [asv.reshape(-1, 1), adv.reshape(-1, 1), mW,
                 mb.reshape(1, -1)]
        out_specs = [pl.BlockSpec((tm, HW), lambda i: (i, 0)),
                     pl.BlockSpec((tm, HW), lambda i: (i, 0)),
                     pl.BlockSpec((tm, 1), lambda i: (i, 0)),
                     pl.BlockSpec((tm, 1), lambda i: (i, 0))]
        out_shape = [jax.ShapeDtypeStruct((N, HW), F32),
                     jax.ShapeDtypeStruct((N, HW), F32),
                     jax.ShapeDtypeStruct((N, 1), F32),
                     jax.ShapeDtypeStruct((N, 1), F32)]
    o = pl.pallas_call(body, grid=(N // tm,), in_specs=in_specs,
                       out_specs=out_specs, out_shape=out_shape,
                       interpret=_INTERPRET)(*args)
    if nxt is None:
        return o
    return o[0], o[1], o[2].reshape(N), o[3].reshape(N)


def _seg_sum_mat(V, segf, Gn, tm):
    N, D = V.shape

    def body(v_ref, s_ref, o_ref):
        i = pl.program_id(0)

        @pl.when(i == 0)
        def _():
            o_ref[...] = jnp.zeros_like(o_ref)

        oh = (s_ref[...] == lax.broadcasted_iota(jnp.int32, (tm, Gn), 1).astype(F32)).astype(F32)
        o_ref[...] += lax.dot_general(oh, v_ref[...], (((0,), (0,)), ((), ())),
                                      preferred_element_type=F32,
                                      precision=lax.Precision.HIGHEST)

    return pl.pallas_call(
        body, grid=(N // tm,),
        in_specs=[pl.BlockSpec((tm, D), lambda i: (i, 0)),
                  pl.BlockSpec((tm, 1), lambda i: (i, 0))],
        out_specs=pl.BlockSpec((Gn, D), lambda i: (0, 0)),
        out_shape=jax.ShapeDtypeStruct((Gn, D), F32),
        interpret=_INTERPRET,
    )(V, segf)


def _mol2(h2, segf, g0, a1, a2, mW, mb, Gn, tm):
    N, D = h2.shape

    def body(h_ref, s_ref, g_ref, a1r, a2r, mwr, mbr, z_ref, w_ref):
        i = pl.program_id(0)

        @pl.when(i == 0)
        def _():
            z_ref[...] = jnp.zeros_like(z_ref)
            w_ref[...] = jnp.zeros_like(w_ref)

        oh = (s_ref[...] == lax.broadcasted_iota(jnp.int32, (tm, Gn), 1).astype(F32)).astype(F32)
        ht = h_ref[...]
        gseg = _dotx(oh, g_ref[...])
        sc = _lrelu(_dot(ht, a1r[...]) + _dot(gseg, a2r[...]))
        e = jnp.exp(sc)
        mh = _dot(ht, mwr[...]) + mbr[...]
        z_ref[...] += lax.dot_general(oh, e, (((0,), (0,)), ((), ())),
                                      preferred_element_type=F32,
                                      precision=lax.Precision.HIGHEST)
        w_ref[...] += lax.dot_general(oh, e * mh, (((0,), (0,)), ((), ())),
                                      preferred_element_type=F32,
                                      precision=lax.Precision.HIGHEST)

    return pl.pallas_call(
        body, grid=(N // tm,),
        in_specs=[pl.BlockSpec((tm, D), lambda i: (i, 0)),
                  pl.BlockSpec((tm, 1), lambda i: (i, 0)),
                  pl.BlockSpec((Gn, D), lambda i: (0, 0)),
                  pl.BlockSpec((D, 1), lambda i: (0, 0)),
                  pl.BlockSpec((D, 1), lambda i: (0, 0)),
                  pl.BlockSpec(mW.shape, lambda i: (0, 0)),
                  pl.BlockSpec((1, D), lambda i: (0, 0))],
        out_specs=[pl.BlockSpec((Gn, 1), lambda i: (0, 0)),
                   pl.BlockSpec((Gn, D), lambda i: (0, 0))],
        out_shape=[jax.ShapeDtypeStruct((Gn, 1), F32),
                   jax.ShapeDtypeStruct((Gn, D), F32)],
        interpret=_INTERPRET,
    )(h2, segf, g0, a1.reshape(-1, 1), a2.reshape(-1, 1), mW,
      mb.reshape(1, -1))


def _mol3(g0, zs, ws, outW, outb):
    Gn = g0.shape[0]

    def body(g_ref, z_ref, w_ref, ow, ob, o_ref):
        z = z_ref[...]
        ctx = w_ref[...] / (z + 1e-16)
        g = _elu(g_ref[...] + ctx)
        o_ref[...] = _dot(g, ow[...]) + ob[...]

    return pl.pallas_call(
        body, out_shape=jax.ShapeDtypeStruct((Gn, outW.shape[1]), F32),
        interpret=_INTERPRET,
    )(g0, zs, ws, outW, outb.reshape(1, -1))


def _att_bn(o0, o1, W, b):
    Gn = o0.shape[0]

    def body(r0, r1, wr, br, o_ref):
        cat = jnp.concatenate([r0[...], r1[...]], axis=1)
        y = _dot(cat, wr[...]) + br[...]
        mu = jnp.mean(y, axis=0, keepdims=True)
        var = jnp.mean(y * y, axis=0, keepdims=True) - mu * mu
        o_ref[...] = jnp.maximum((y - mu) * lax.rsqrt(var + 1e-5), 0.0)

    return pl.pallas_call(
        body, out_shape=jax.ShapeDtypeStruct((Gn, W.shape[1]), F32),
        interpret=_INTERPRET,
    )(o0, o1, W, b.reshape(1, -1))


def _jt_embed(gf, motif, mstats, jW, jb, projs, heads):
    Fn = gf.shape[0]

    def body(*refs):
        gfr, mr, st, jwr, jbr = refs[:5]
        rest = refs[5:]
        me = _lrelu(_bn_from_stats(_dot(mr[...], jwr[...]) + jbr[...],
                                   st[...], Fn))
        jx = jnp.concatenate([gfr[...], me], axis=1)
        for h in range(2):
            pjW, pjb = rest[2 * h], rest[2 * h + 1]
            pnW, pnb, asv, adv, mW, mb = rest[4 + 6 * h:10 + 6 * h]
            outs = rest[16 + 4 * h:20 + 4 * h]
            xh = _dot(jx, pjW[...]) + pjb[...]
            h0 = _lrelu(_dot(xh, pnW[...]) + pnb[...])
            outs[0][...] = h0
            outs[1][...] = _dot(h0, mW[...]) + mb[...]
            outs[2][...] = _dot(h0, asv[...])
            outs[3][...] = _dot(h0, adv[...])

    args = [gf, motif, mstats, jW, jb.reshape(1, -1)]
    for (pjW, pjb) in projs:
        args += [pjW, pjb.reshape(1, -1)]
    for (pnW, pnb, asv, adv, mW, mb) in heads:
        args += [pnW, pnb.reshape(1, -1), asv.reshape(-1, 1),
                 adv.reshape(-1, 1), mW, mb.reshape(1, -1)]
    out_shape = []
    for _ in range(2):
        out_shape += [jax.ShapeDtypeStruct((Fn, HW), F32),
                      jax.ShapeDtypeStruct((Fn, HW), F32),
                      jax.ShapeDtypeStruct((Fn, 1), F32),
                      jax.ShapeDtypeStruct((Fn, 1), F32)]
    o = pl.pallas_call(body, out_shape=out_shape, interpret=_INTERPRET)(*args)
    return [(o[0], o[1], o[2].reshape(Fn), o[3].reshape(Fn)),
            (o[4], o[5], o[6].reshape(Fn), o[7].reshape(Fn))]


def _final(go, fp, gj0, gj1, p1W, p1b, p2aW, p2ab, p2bW, p2bb):
    Gn = go.shape[0]

    def body(g0r, fpr, j0r, j1r, w1, b1, w2, b2, w3, b3, o_ref):
        sng = jnp.maximum((j0r[...] + j1r[...]) * 0.5, 0.0)
        cat = jnp.concatenate([g0r[...], fpr[...], sng], axis=1)
        y = _lrelu(_dot(cat, w1[...]) + b1[...], 0.001)
        mu = jnp.mean(y, axis=0, keepdims=True)
        var = jnp.mean(y * y, axis=0, keepdims=True) - mu * mu
        h = (y - mu) * lax.rsqrt(var + 1e-5)
        h = _lrelu(_dot(h, w2[...]) + b2[...], 0.001)
        o_ref[...] = _dot(h, w3[...]) + b3[...]

    return pl.pallas_call(
        body, out_shape=jax.ShapeDtypeStruct((Gn, 1), F32),
        interpret=_INTERPRET,
    )(go, fp, gj0, gj1, p1W, p1b.reshape(1, -1), p2aW, p2ab.reshape(1, -1),
      p2bW, p2bb.reshape(1, -1))


def _edge_pass_sc(hext, u, v, w, src, dst, interpret=False):
    N = hext.shape[0]
    EP = w.shape[0]
    per_tile = EP // SC_WORKERS
    nchunks = per_tile // SC_CHUNK
    npt_main = ((N // 16 + 7) // 8) * 8
    npt_last = N - 15 * npt_main
    mesh = plsc.VectorSubcoreMesh(core_axis_name="c", subcore_axis_name="s",
                                  num_cores=2, num_subcores=16)

    def _chunks(total):
        out, off = [], 0
        while off < total:
            sz = min(128, total - off)
            out.append((off, sz))
            off += sz
        return out

    chunks_main = _chunks(npt_main)
    chunks_last = _chunks(npt_last)

    @functools.partial(
        pl.kernel, mesh=mesh,
        out_type=(jax.ShapeDtypeStruct((2, N, HW), F32),
                  jax.ShapeDtypeStruct((SC_WORKERS, N), F32)),
        scratch_types=[
            pltpu.VMEM((N,), F32),
            pltpu.VMEM((N,), F32),
            pltpu.VMEM((N,), F32),
            pltpu.VMEM((SC_CHUNK,), jnp.int32),
            pltpu.VMEM((SC_CHUNK,), jnp.int32),
            pltpu.VMEM((SC_CHUNK,), F32),
            pltpu.VMEM((SC_CHUNK,), F32),
            pltpu.VMEM((SC_CHUNK, HW), F32),
            pltpu.VMEM_SHARED((N, HW), F32),
            pltpu.SemaphoreType.DMA,
        ],
        compiler_params=pltpu.CompilerParams(needs_layout_passes=False),
        interpret=interpret,
    )
    def k(hext_h, u_h, v_h, w_h, src_h, dst_h, out_h, zout_h,
          u_vm, v_vm, z_vm, src_vm, dst_vm, w_vm, c_vm, rows_vm, acc_sh, sem):
        ci = lax.axis_index("c")
        si = lax.axis_index("s")
        wid = si * 2 + ci
        base = wid * per_tile
        rbase = si * npt_main

        def zrow(r, _):
            for j in range(HW // 16):
                rows_vm[r, pl.ds(j * 16, 16)] = jnp.zeros((16,), F32)
            return 0

        lax.fori_loop(0, SC_CHUNK, zrow, 0)

        def zzero(r, _):
            z_vm[pl.ds(r * 16, 16)] = jnp.zeros((16,), F32)
            return 0

        lax.fori_loop(0, N // 16, zzero, 0)

        @pl.when(si < 15)
        def _():
            for (roff, rsz) in chunks_main:
                pltpu.sync_copy(rows_vm.at[pl.ds(0, rsz)],
                                acc_sh.at[pl.ds(rbase + roff, rsz)])

        @pl.when(si == 15)
        def _():
            for (roff, rsz) in chunks_last:
                pltpu.sync_copy(rows_vm.at[pl.ds(0, rsz)],
                                acc_sh.at[pl.ds(rbase + roff, rsz)])
        pltpu.sync_copy(u_h, u_vm)
        pltpu.sync_copy(v_h, v_vm)
        plsc.subcore_barrier()

        def chunk(cidx, _):
            off = base + cidx * SC_CHUNK
            pltpu.sync_copy(src_h.at[pl.ds(off, SC_CHUNK)], src_vm)
            pltpu.sync_copy(dst_h.at[pl.ds(off, SC_CHUNK)], dst_vm)
            pltpu.sync_copy(w_h.at[pl.ds(off, SC_CHUNK)], w_vm)
            pltpu.async_copy(hext_h.at[src_vm], rows_vm, sem).wait()
            lanes = lax.iota(jnp.int32, 16)
            for i in range(SC_CHUNK // 16):
                sl = pl.ds(i * 16, 16)
                dsts = dst_vm[sl]
                us = plsc.load_gather(u_vm, [src_vm[sl]])
                vs = plsc.load_gather(v_vm, [dsts])
                s = us + vs + w_vm[sl]
                s = jnp.where(s >= 0, s, 0.01 * s)
                cvec = jnp.exp(s)
                c_vm[sl] = cvec
                for j in range(16):
                    plsc.addupdate_scatter(z_vm, [dsts], cvec,
                                           mask=lanes == j)

            def scale(r, _):
                cs = plsc.load_gather(c_vm, [jnp.full((16,), r, jnp.int32)])
                for j in range(HW // 16):
                    rsl = pl.ds(j * 16, 16)
                    rows_vm[r, rsl] = rows_vm[r, rsl] * cs
                return 0

            lax.fori_loop(0, SC_CHUNK, scale, 0)
            pltpu.sync_copy(rows_vm, acc_sh.at[dst_vm], add=True)
            return 0

        lax.fori_loop(0, nchunks, chunk, 0)
        pltpu.sync_copy(z_vm, zout_h.at[wid])
        plsc.subcore_barrier()

        @pl.when(si < 15)
        def _():
            for (roff, rsz) in chunks_main:
                pltpu.sync_copy(acc_sh.at[pl.ds(rbase + roff, rsz)],
                                out_h.at[ci, pl.ds(rbase + roff, rsz)])

        @pl.when(si == 15)
        def _():
            for (roff, rsz) in chunks_last:
                pltpu.sync_copy(acc_sh.at[pl.ds(rbase + roff, rsz)],
                                out_h.at[ci, pl.ds(rbase + roff, rsz)])

    return k(hext, u, v, w, src, dst)


def _pad_edges(src, dst, w):
    E = src.shape[0]
    EP = ((E + 4095) // 4096) * 4096
    if EP == E:
        return src, dst, w
    pad = EP - E
    src = jnp.concatenate([src, jnp.zeros((pad,), jnp.int32)])
    dst = jnp.concatenate([dst, jnp.zeros((pad,), jnp.int32)])
    w = jnp.concatenate([w, jnp.full((pad,), -1e30, F32)])
    return src, dst, w


def _afp_channel(hp_list, tables, src, dst, w_lists, segf, Gn, n, tm, mol_tm):
    src = src.astype(jnp.int32)
    dst = dst.astype(jnp.int32)
    outs = []
    for h, hp in enumerate(hp_list):
        hh, tab, uu, vv = tables[h]
        for l in range(len(hp['atom'])):
            sp, dp, wp = _pad_edges(src, dst, w_lists[h][l])
            acc, zp = _edge_pass_sc(tab, uu, vv, wp, sp, dp,
                                    interpret=_INTERPRET)
            zp = zp.T
            if l + 1 < len(hp['atom']):
                nl = hp['atom'][l + 1]
                hh, tab, uu, vv = _node_update(
                    hh, acc, zp,
                    (nl['as'], nl['ad'], nl['msg']['W'], nl['msg']['b']), tm)
            else:
                h2 = _node_update(hh, acc, zp, None, tm)
        ml = hp['mol'][0]
        g0 = _seg_sum_mat(h2, segf, Gn, mol_tm)
        zs, ws = _mol2(h2, segf, g0, ml['a1'], ml['a2'], ml['msg']['W'],
                       ml['msg']['b'], Gn, mol_tm)
        outs.append(_mol3(g0, zs, ws, hp['out']['W'], hp['out']['b']))
    return outs


def kernel(x, edge_index, edge_attr, batch, frag_x, frag_edge_index,
           frag_edge_attr, frag_batch, junction_edge_index, junction_edge_attr,
           junction_batch, motif_nodes, params):
    p = params
    N, E = x.shape[0], edge_attr.shape[0]
    NF, EF = frag_x.shape[0], frag_edge_attr.shape[0]
    Fn, EJ = motif_nodes.shape[0], junction_edge_attr.shape[0]
    G = 256

    batchf = batch.astype(F32).reshape(-1, 1)
    fbatchf = frag_batch.astype(F32).reshape(-1, 1)
    jbatchf = junction_batch.astype(F32).reshape(-1, 1)

    xst = _stats(x, p['o_node']['W'], p['o_node']['b'], 2000)
    east = _stats(edge_attr, p['o_edge']['W'], p['o_edge']['b'], 3200)
    ohp = [(hp['pe']['W'], hp['pe']['b'], hp['atom'][0]['ae'],
            hp['atom'][1]['ae']) for hp in p['o_afp']]
    ow = _echain(edge_attr, p['o_edge']['W'], p['o_edge']['b'], east, E,
                 ohp, 3200)
    ow = [[ow[0], ow[1]], [ow[2], ow[3]]]
    oheads = [(hp['pn']['W'], hp['pn']['b'], hp['atom'][0]['as'],
               hp['atom'][0]['ad'], hp['atom'][0]['msg']['W'],
               hp['atom'][0]['msg']['b']) for hp in p['o_afp']]
    otab = _embed_heads(x, p['o_node']['W'], p['o_node']['b'], xst, N,
                        oheads, 2000)
    o_outs = _afp_channel(p['o_afp'], otab, edge_index[0], edge_index[1],
                          ow, batchf, G, N, 2000, 2000)
    graph_origin = _att_bn(o_outs[0], o_outs[1], p['o_att']['W'],
                           p['o_att']['b'])

    fhp = [(hp['pe']['W'], hp['pe']['b'], hp['atom'][0]['ae'],
            hp['atom'][1]['ae']) for hp in p['f_afp']]
    fw = _echain(frag_edge_attr, None, None, None, EF, fhp, 3200)
    fw = [[fw[0], fw[1]], [fw[2], fw[3]]]
    fheads = [(hp['pn']['W'], hp['pn']['b'], hp['atom'][0]['as'],
               hp['atom'][0]['ad'], hp['atom'][0]['msg']['W'],
               hp['atom'][0]['msg']['b']) for hp in p['f_afp']]
    ftab = _embed_heads(frag_x, None, None, None, NF, fheads, 2000)
    f_outs = _afp_channel(p['f_afp'], ftab, frag_edge_index[0],
                          frag_edge_index[1], fw, fbatchf, Fn, NF, 2000, 1000)
    graph_frag = _att_bn(f_outs[0], f_outs[1], p['f_att']['W'],
                         p['f_att']['b'])

    mst = _stats(motif_nodes, p['j_frag']['W'], p['j_frag']['b'], 2000)
    jest = _stats(junction_edge_attr, p['j_edge']['W'], p['j_edge']['b'], 8000)
    jhp = [(hp['pe']['W'], hp['pe']['b'], hp['atom'][0]['ae'],
            hp['atom'][1]['ae']) for hp in p['j_afp']]
    jw = _echain(junction_edge_attr, p['j_edge']['W'], p['j_edge']['b'],
                 jest, EJ, jhp, 8000)
    jw = [[jw[0], jw[1]], [jw[2], jw[3]]]
    jheads = [(hp['pn']['W'], hp['pn']['b'], hp['atom'][0]['as'],
               hp['atom'][0]['ad'], hp['atom'][0]['msg']['W'],
               hp['atom'][0]['msg']['b']) for hp in p['j_afp']]
    jtab = _jt_embed(graph_frag, motif_nodes, mst, p['j_frag']['W'],
                     p['j_frag']['b'],
                     [(pr['W'], pr['b']) for pr in p['j_proj']], jheads)
    j_outs = _afp_channel(p['j_afp'], jtab, junction_edge_index[0],
                          junction_edge_index[1], jw, jbatchf, G, Fn, 2000,
                          2000)

    frag_pooled = _seg_sum_mat(graph_frag, jbatchf, G, 2000)
    return _final(graph_origin, frag_pooled, j_outs[0], j_outs[1],
                  p['p1']['W'], p['p1']['b'], p['p2a']['W'], p['p2a']['b'],
                  p['p2b']['W'], p['p2b']['b'])
